# baseline (speedup 1.0000x reference)
import functools

import jax
import jax.numpy as jnp
from jax import lax
from jax.experimental import pallas as pl
from jax.experimental.pallas import tpu as pltpu
from jax.experimental.pallas import tpu_sc as plsc

_N_CHARS = 1000
_EMB = 32
_BATCH = 4096
_SEQ = 200
_TOT = _BATCH * _SEQ

_NC = 2
_NS = 16
_NW = _NC * _NS
_L = 16
_ER = _EMB // 8
_BC = _BATCH // 128
_SPW = _SEQ // (_NW // _ER)
_UNIT = 8 * _BATCH


def _make_gather():
    mesh = plsc.VectorSubcoreMesh(core_axis_name="c", subcore_axis_name="s")

    @functools.partial(
        pl.kernel,
        mesh=mesh,
        out_type=jax.ShapeDtypeStruct((_TOT * _EMB,), jnp.float32),
        scratch_types=[
            pltpu.VMEM((2, _BC, 128), jnp.int32),
            pltpu.VMEM((2, _UNIT), jnp.float32),
            pltpu.VMEM((_EMB * _N_CHARS,), jnp.float32),
            pltpu.SemaphoreType.DMA,
            pltpu.SemaphoreType.DMA,
            pltpu.SemaphoreType.DMA,
            pltpu.SemaphoreType.DMA,
        ],
        compiler_params=pltpu.CompilerParams(
            use_tc_tiling_on_sc=False, needs_layout_passes=False),
    )
    def gather_kernel(table_t_hbm, idx_hbm, out_hbm,
                      idx_v, unit_v, table_v, si0, si1, so0, so1):
        sem_i = (si0, si1)
        sem_o = (so0, so1)
        wid = lax.axis_index("s") * _NC + lax.axis_index("c")
        er = wid // (_NW // _ER)
        s0 = (wid % (_NW // _ER)) * _SPW

        pltpu.sync_copy(table_t_hbm, table_v)

        def idx_copy(u, b):
            s = s0 + u
            return pltpu.make_async_copy(
                idx_hbm.at[s // 8, :, s % 8, :], idx_v.at[b], sem_i[b])

        def out_copy(u, b):
            base = pl.multiple_of(
                (s0 + u) * (_EMB * _BATCH) + er * _UNIT, 8)
            return pltpu.make_async_copy(
                unit_v.at[b], out_hbm.at[pl.ds(base, _UNIT)], sem_o[b])

        ebase = er * 8 * _N_CHARS

        def compute_unit(b):
            idx_ref = idx_v.at[b]
            buf = unit_v.at[b]

            @plsc.parallel_loop(0, _BC * (128 // _L), unroll=2)
            def blk(t):
                bc = t // 8
                k = t % 8
                idx16 = idx_ref[bc, pl.ds(k * _L, _L)]
                off = bc * 1024 + k * _L
                for el in range(8):
                    a = idx16 + (ebase + el * _N_CHARS)
                    v = plsc.load_gather(table_v, [a])
                    buf[pl.ds(off + el * 128, _L)] = v

        idx_copy(0, 0).start()
        idx_copy(1, 1).start()
        for b in range(2):
            idx_copy(b, b).wait()
            compute_unit(b)
            out_copy(b, b).start()
            idx_copy(b + 2, b).start()

        def steady(g, carry):
            for b in range(2):
                u = 2 + 2 * g + b
                idx_copy(u, b).wait()
                out_copy(u - 2, b).wait()
                compute_unit(b)
                out_copy(u, b).start()

                @pl.when(u + 2 < _SPW)
                def _():
                    idx_copy(u + 2, b).start()
            return carry

        lax.fori_loop(0, (_SPW - 3) // 2, steady, 0)

        u_last = _SPW - 1
        idx_copy(u_last, 0).wait()
        out_copy(u_last - 2, 0).wait()
        compute_unit(0)
        out_copy(u_last, 0).start()
        out_copy(u_last - 1, 1).wait()
        out_copy(u_last, 0).wait()

    return gather_kernel


_gather = _make_gather()


@jax.jit
def kernel(sentence, table):
    idx_t = sentence.reshape(_BC, 128, _SEQ // 8, 8).transpose(2, 0, 3, 1)
    table_t = table.T.reshape(_EMB * _N_CHARS)
    out = _gather(table_t, idx_t)
    t = out.reshape(_SEQ, _ER, _BC, 8, 128)
    t = t.transpose(2, 4, 0, 1, 3)
    return t.reshape(_BATCH, _SEQ, _EMB)


# speedup vs baseline: 1.0061x; 1.0061x over previous
import functools

import jax
import jax.numpy as jnp
from jax import lax
from jax.experimental import pallas as pl
from jax.experimental.pallas import tpu as pltpu
from jax.experimental.pallas import tpu_sc as plsc

_N_CHARS = 1000
_EMB = 32
_BATCH = 4096
_SEQ = 200
_TOT = _BATCH * _SEQ

_NC = 2
_NS = 16
_NW = _NC * _NS
_L = 16
_ER = _EMB // 8
_BC = _BATCH // 128
_SPW = _SEQ // (_NW // _ER)
_UNIT = 8 * _BATCH


def _make_gather():
    mesh = plsc.VectorSubcoreMesh(core_axis_name="c", subcore_axis_name="s")

    @functools.partial(
        pl.kernel,
        mesh=mesh,
        out_type=jax.ShapeDtypeStruct((_TOT * _EMB,), jnp.float32),
        scratch_types=[
            pltpu.VMEM((2, _BC, 128), jnp.int32),
            pltpu.VMEM((2, _UNIT), jnp.float32),
            pltpu.VMEM((_EMB * _N_CHARS,), jnp.float32),
            pltpu.SemaphoreType.DMA,
            pltpu.SemaphoreType.DMA,
            pltpu.SemaphoreType.DMA,
            pltpu.SemaphoreType.DMA,
            pltpu.SemaphoreType.DMA,
        ],
        compiler_params=pltpu.CompilerParams(
            use_tc_tiling_on_sc=False, needs_layout_passes=False),
    )
    def gather_kernel(table_t_hbm, idx_hbm, out_hbm,
                      idx_v, unit_v, table_v, si0, si1, so0, so1, st):
        sem_i = (si0, si1)
        sem_o = (so0, so1)
        wid = lax.axis_index("s") * _NC + lax.axis_index("c")
        er = wid // (_NW // _ER)
        s0 = (wid % (_NW // _ER)) * _SPW

        table_cp = pltpu.make_async_copy(table_t_hbm, table_v, st)
        table_cp.start()

        def idx_copy(u, b):
            s = s0 + u
            return pltpu.make_async_copy(
                idx_hbm.at[s // 8, :, s % 8, :], idx_v.at[b], sem_i[b])

        def out_copy(u, b):
            base = pl.multiple_of(
                (s0 + u) * (_EMB * _BATCH) + er * _UNIT, 8)
            return pltpu.make_async_copy(
                unit_v.at[b], out_hbm.at[pl.ds(base, _UNIT)], sem_o[b])

        ebase = er * 8 * _N_CHARS

        def compute_unit(b):
            idx_ref = idx_v.at[b]
            buf = unit_v.at[b]

            @plsc.parallel_loop(0, _BC * (128 // _L), unroll=1)
            def blk(t):
                bc = t // 8
                k = t % 8
                idx16 = idx_ref[bc, pl.ds(k * _L, _L)]
                off = bc * 1024 + k * _L
                for el in range(8):
                    a = idx16 + (ebase + el * _N_CHARS)
                    v = plsc.load_gather(table_v, [a])
                    buf[pl.ds(off + el * 128, _L)] = v

        idx_copy(0, 0).start()
        idx_copy(1, 1).start()
        table_cp.wait()
        for b in range(2):
            idx_copy(b, b).wait()
            compute_unit(b)
            out_copy(b, b).start()
            idx_copy(b + 2, b).start()

        def steady(g, carry):
            for b in range(2):
                u = 2 + 2 * g + b
                idx_copy(u, b).wait()
                out_copy(u - 2, b).wait()
                compute_unit(b)
                out_copy(u, b).start()

                @pl.when(u + 2 < _SPW)
                def _():
                    idx_copy(u + 2, b).start()
            return carry

        lax.fori_loop(0, (_SPW - 3) // 2, steady, 0)

        u_last = _SPW - 1
        idx_copy(u_last, 0).wait()
        out_copy(u_last - 2, 0).wait()
        compute_unit(0)
        out_copy(u_last, 0).start()
        out_copy(u_last - 1, 1).wait()
        out_copy(u_last, 0).wait()

    return gather_kernel


_gather = _make_gather()


@jax.jit
def kernel(sentence, table):
    idx_t = sentence.reshape(_BC, 128, _SEQ // 8, 8).transpose(2, 0, 3, 1)
    table_t = table.T.reshape(_EMB * _N_CHARS)
    out = _gather(table_t, idx_t)
    t = out.reshape(_SEQ, _ER, _BC, 8, 128)
    t = t.transpose(2, 4, 0, 1, 3)
    return t.reshape(_BATCH, _SEQ, _EMB)
